# baseline (speedup 1.0000x reference)
import functools

import jax
import jax.numpy as jnp
from jax import lax
from jax.experimental import pallas as pl
from jax.experimental.pallas import tpu as pltpu
from jax.experimental.pallas import tpu_sc as plsc

_NC = 2
_NS = 16
_NW = _NC * _NS


def _sc_pool(idx2, W2, batch, bag, dim):
    bpw = batch // _NW
    half = dim // 2
    mesh = plsc.VectorSubcoreMesh(core_axis_name="c", subcore_axis_name="s")

    @functools.partial(
        pl.kernel,
        out_type=jax.ShapeDtypeStruct((batch, dim), jnp.float32),
        mesh=mesh,
        scratch_types=[
            pltpu.VMEM((2 * bag, bpw), jnp.int32),
            pltpu.VMEM((bpw, half), jnp.float32),
            pltpu.VMEM((bpw, half), jnp.float32),
            pltpu.SemaphoreType.DMA,
        ],
        compiler_params=pltpu.CompilerParams(use_tc_tiling_on_sc=False),
    )
    def body(idx2_hbm, w2_hbm, out_hbm, idx_v, acc_a, acc_b, sem):
        wid = lax.axis_index("s") * _NC + lax.axis_index("c")
        base = wid * bpw

        pltpu.sync_copy(idx2_hbm.at[:, pl.ds(base, bpw)], idx_v)

        zeros = jnp.zeros((16,), jnp.float32)

        @pl.loop(0, bpw)
        def _(i):
            acc_a[i, :] = zeros
            acc_b[i, :] = zeros

        copies = [
            pltpu.async_copy(w2_hbm.at[idx_v.at[j]], acc_a, sem, add=True)
            for j in range(bag)
        ] + [
            pltpu.async_copy(w2_hbm.at[idx_v.at[bag + j]], acc_b, sem, add=True)
            for j in range(bag)
        ]
        for c in copies:
            c.wait()

        pltpu.sync_copy(acc_a, out_hbm.at[pl.ds(base, bpw), pl.ds(0, half)])
        pltpu.sync_copy(acc_b, out_hbm.at[pl.ds(base, bpw), pl.ds(half, half)])

    return body(idx2, W2)


def _sc_relayout(WT, num_emb, dim):
    rblk = 1024
    cblk = rblk * 32
    grid = (num_emb + cblk - 1) // cblk
    rows = grid * rblk
    mesh = plsc.VectorSubcoreMesh(core_axis_name="c", subcore_axis_name="s")

    @functools.partial(
        pl.kernel,
        out_type=jax.ShapeDtypeStruct((rows, 32 * dim), jnp.float32),
        mesh=mesh,
        scratch_types=[
            pltpu.VMEM((dim, rblk), jnp.float32),
            pltpu.VMEM((rblk * dim // 1024, 1024), jnp.float32),
        ],
        compiler_params=pltpu.CompilerParams(
            use_tc_tiling_on_sc=False, needs_layout_passes=False),
    )
    def body(wt_hbm, out_hbm, sin, sout):
        wid = lax.axis_index("s") * _NC + lax.axis_index("c")
        lanes = lax.iota(jnp.int32, 16)

        @pl.loop(0, grid)
        def _(i):
            c0 = i * cblk + wid * rblk

            @pl.when(c0 + rblk <= num_emb)
            def _():
                pltpu.sync_copy(wt_hbm.at[:, pl.ds(c0, rblk)], sin)

                @pl.loop(0, rblk // 16)
                def _(c16):
                    cs = c16 * 16
                    cols = cs + lanes
                    ro = cols >> 5
                    co = (cols & 31) << 5
                    for d in range(dim):
                        v = sin[d, pl.ds(cs, 16)]
                        plsc.store_scatter(sout, [ro, co + d], v)

                pltpu.sync_copy(
                    sout,
                    out_hbm.at[pl.ds((wid * grid + i) * (rblk * dim // 1024),
                                     rblk * dim // 1024), :])

        tail_start = (num_emb // rblk) * rblk
        tail_len = num_emb - tail_start
        if tail_len:
            t_i = tail_start // cblk
            t_k = (tail_start // rblk) % 32

            @pl.when(wid == t_k)
            def _():
                pltpu.sync_copy(wt_hbm.at[:, pl.ds(tail_start, tail_len)],
                                sin.at[:, pl.ds(0, tail_len)])

                @pl.loop(0, tail_len // 16)
                def _(c16):
                    cs = c16 * 16
                    cols = cs + lanes
                    ro = cols >> 5
                    co = (cols & 31) << 5
                    for d in range(dim):
                        v = sin[d, pl.ds(cs, 16)]
                        plsc.store_scatter(sout, [ro, co + d], v)

                trows = tail_len * dim // 1024
                pltpu.sync_copy(
                    sout.at[pl.ds(0, trows)],
                    out_hbm.at[pl.ds((t_k * grid + t_i) * (rblk * dim // 1024),
                                     trows), :])

    return body(WT)


def _tc_relayout(WT, num_emb, dim):
    rblk = 1024
    cblk = rblk * 32
    grid = (num_emb + cblk - 1) // cblk
    rows = grid * rblk

    def body(x_ref, o_ref):
        pieces = [
            x_ref[:, k * rblk:(k + 1) * rblk].T for k in range(32)
        ]
        o_ref[...] = jnp.concatenate(pieces, axis=1)

    return pl.pallas_call(
        body,
        grid=(grid,),
        in_specs=[pl.BlockSpec((dim, cblk), lambda i: (0, i))],
        out_specs=pl.BlockSpec((rblk, 32 * dim), lambda i: (i, 0)),
        out_shape=jax.ShapeDtypeStruct((rows, 32 * dim), jnp.float32),
    )(WT)


def _tc_proj(pooled, P, batch, dim, out_dim):
    blk = 1024

    def body(x_ref, p_ref, o_ref):
        o_ref[...] = lax.dot_general(
            x_ref[...], p_ref[...],
            (((1,), (1,)), ((), ())),
            preferred_element_type=jnp.float32,
        )

    return pl.pallas_call(
        body,
        grid=(batch // blk,),
        in_specs=[
            pl.BlockSpec((blk, dim), lambda i: (i, 0)),
            pl.BlockSpec((out_dim, dim), lambda i: (0, 0)),
        ],
        out_specs=pl.BlockSpec((blk, out_dim), lambda i: (i, 0)),
        out_shape=jax.ShapeDtypeStruct((batch, out_dim), jnp.float32),
    )(pooled, P)


def kernel(input, W, P):
    batch, bag = input.shape
    num_emb, dim = W.shape
    out_dim = P.shape[0]
    W_w = _sc_relayout(W.T, num_emb, dim)
    W2 = W_w.reshape(W_w.shape[0] * W_w.shape[1] // (dim // 2), dim // 2)

    grid = (num_emb + 32767) // 32768
    e = input.astype(jnp.int32)
    i = e >> 15
    k = (e >> 10) & 31
    r = e & 1023
    row_a = ((k * grid + i) << 11) | (r << 1)
    idx2 = jnp.concatenate([row_a.T, (row_a | 1).T], axis=0)

    pooled = _sc_pool(idx2, W2, batch, bag, dim)
    return _tc_proj(pooled, P, batch, dim, out_dim)


# speedup vs baseline: 11.5293x; 11.5293x over previous
import functools

import jax
import jax.numpy as jnp
from jax import lax
from jax.experimental import pallas as pl
from jax.experimental.pallas import tpu as pltpu
from jax.experimental.pallas import tpu_sc as plsc

_NC = 2
_NS = 16
_NW = _NC * _NS


def _sc_pool(idx2, W2, batch, bag, dim):
    bpw = batch // _NW
    half = dim // 2
    mesh = plsc.VectorSubcoreMesh(core_axis_name="c", subcore_axis_name="s")

    @functools.partial(
        pl.kernel,
        out_type=jax.ShapeDtypeStruct((batch, dim), jnp.float32),
        mesh=mesh,
        scratch_types=[
            pltpu.VMEM((2 * bag, bpw), jnp.int32),
            pltpu.VMEM((bpw, half), jnp.float32),
            pltpu.VMEM((bpw, half), jnp.float32),
            pltpu.SemaphoreType.DMA,
        ],
        compiler_params=pltpu.CompilerParams(use_tc_tiling_on_sc=False),
    )
    def body(idx2_hbm, w2_hbm, out_hbm, idx_v, acc_a, acc_b, sem):
        wid = lax.axis_index("s") * _NC + lax.axis_index("c")
        base = wid * bpw

        pltpu.sync_copy(idx2_hbm.at[:, pl.ds(base, bpw)], idx_v)

        zeros = jnp.zeros((16,), jnp.float32)

        @pl.loop(0, bpw)
        def _(i):
            acc_a[i, :] = zeros
            acc_b[i, :] = zeros

        copies = [
            pltpu.async_copy(w2_hbm.at[idx_v.at[j]], acc_a, sem, add=True)
            for j in range(bag)
        ] + [
            pltpu.async_copy(w2_hbm.at[idx_v.at[bag + j]], acc_b, sem, add=True)
            for j in range(bag)
        ]
        for c in copies:
            c.wait()

        pltpu.sync_copy(acc_a, out_hbm.at[pl.ds(base, bpw), pl.ds(0, half)])
        pltpu.sync_copy(acc_b, out_hbm.at[pl.ds(base, bpw), pl.ds(half, half)])

    return body(idx2, W2)


def _tc_relayout(WT, num_emb, dim):
    rblk = 1024
    cblk = rblk * 32
    grid = (num_emb + cblk - 1) // cblk
    rows = grid * rblk

    def body(x_ref, o_ref):
        pieces = [
            x_ref[:, k * rblk:(k + 1) * rblk].T for k in range(32)
        ]
        o_ref[...] = jnp.concatenate(pieces, axis=1)

    return pl.pallas_call(
        body,
        grid=(grid,),
        in_specs=[pl.BlockSpec((dim, cblk), lambda i: (0, i))],
        out_specs=pl.BlockSpec((rblk, 32 * dim), lambda i: (i, 0)),
        out_shape=jax.ShapeDtypeStruct((rows, 32 * dim), jnp.float32),
    )(WT)


def _tc_proj(pooled, P, batch, dim, out_dim):
    blk = 1024

    def body(x_ref, p_ref, o_ref):
        o_ref[...] = lax.dot_general(
            x_ref[...], p_ref[...],
            (((1,), (1,)), ((), ())),
            preferred_element_type=jnp.float32,
        )

    return pl.pallas_call(
        body,
        grid=(batch // blk,),
        in_specs=[
            pl.BlockSpec((blk, dim), lambda i: (i, 0)),
            pl.BlockSpec((out_dim, dim), lambda i: (0, 0)),
        ],
        out_specs=pl.BlockSpec((blk, out_dim), lambda i: (i, 0)),
        out_shape=jax.ShapeDtypeStruct((batch, out_dim), jnp.float32),
    )(pooled, P)


def kernel(input, W, P):
    batch, bag = input.shape
    num_emb, dim = W.shape
    out_dim = P.shape[0]
    W_w = _tc_relayout(W.T, num_emb, dim)
    nw = W_w.shape[0] // 8
    W2 = (W_w.reshape(nw, 8, 8, 128)
             .transpose(0, 2, 1, 3)
             .reshape(nw * 8 * 8 * 128 // (dim // 2), dim // 2))

    e = input.astype(jnp.int32)
    i = e >> 15
    k = (e >> 10) & 31
    r = e & 1023
    a = (i << 7) | (r >> 3)
    row_a = (((((a << 3) | (k >> 2)) << 3) | (r & 7)) << 3) | ((k & 3) << 1)
    idx2 = jnp.concatenate([row_a.T, (row_a | 1).T], axis=0)

    pooled = _sc_pool(idx2, W2, batch, bag, dim)
    return _tc_proj(pooled, P, batch, dim, out_dim)
